# baseline (speedup 1.0000x reference)
import functools

import jax
import jax.numpy as jnp
from jax import lax
from jax.experimental import pallas as pl
from jax.experimental.pallas import tpu as pltpu
from jax.experimental.pallas import tpu_sc as plsc

_MU = 3.5
_CHUNK = 32


@functools.lru_cache(maxsize=None)
def _build_sc_kernel(B, K):
    info = plsc.get_sparse_core_info()
    NC, NS, L = info.num_cores, info.num_subcores, info.num_lanes
    NW = NC * NS
    assert B % (8 * NW) == 0 and K % L == 0
    b_per_w = B // NW
    n_chunks = b_per_w // _CHUNK
    mesh = plsc.VectorSubcoreMesh(core_axis_name="c", subcore_axis_name="s",
                                  num_cores=NC, num_subcores=NS)

    @functools.partial(
        pl.kernel,
        out_type=jax.ShapeDtypeStruct((B,), jnp.float32),
        mesh=mesh,
        scratch_types=[
            pltpu.VMEM((b_per_w,), jnp.int32),
            pltpu.VMEM((b_per_w,), jnp.int32),
            pltpu.VMEM((_CHUNK, K), jnp.float32),
            pltpu.VMEM((_CHUNK, K), jnp.float32),
            pltpu.VMEM((b_per_w,), jnp.float32),
            pltpu.VMEM((b_per_w,), jnp.float32),
            pltpu.VMEM((b_per_w,), jnp.float32),
            pltpu.SemaphoreType.DMA,
            pltpu.SemaphoreType.DMA,
            pltpu.SemaphoreType.DMA,
            pltpu.SemaphoreType.DMA,
        ],
        compiler_params=pltpu.CompilerParams(needs_layout_passes=False),
        interpret=False,
    )
    def sc_kernel(uidx_hbm, iidx_hbm, p_hbm, q_hbm, bu_hbm, bi_hbm, out_hbm,
                  idx_u, idx_i, rows_p, rows_q, bu_v, bi_v, out_v,
                  sem0, sem1, sem2, sem3):
        wid = lax.axis_index("s") * NC + lax.axis_index("c")
        base = wid * b_per_w
        for c in range(b_per_w // 128):
            pltpu.sync_copy(uidx_hbm.at[pl.ds(base + c * 128, 128)],
                            idx_u.at[pl.ds(c * 128, 128)])
            pltpu.sync_copy(iidx_hbm.at[pl.ds(base + c * 128, 128)],
                            idx_i.at[pl.ds(c * 128, 128)])

        bias_copies = []
        for c in range(b_per_w // 128):
            sl = pl.ds(c * 128, 128)
            bias_copies.append(
                pltpu.async_copy(bu_hbm.at[idx_u.at[sl]], bu_v.at[sl], sem2))
            bias_copies.append(
                pltpu.async_copy(bi_hbm.at[idx_i.at[sl]], bi_v.at[sl], sem3))

        lane = lax.iota(jnp.int32, L)

        def chunk_body(c, carry):
            cbase = c * _CHUNK

            for g in range(_CHUNK // L):
                u_vec = idx_u[pl.ds(cbase + g * L, L)]
                i_vec = idx_i[pl.ds(cbase + g * L, L)]
                for j in range(L):
                    pltpu.async_copy(
                        p_hbm.at[u_vec[j]], rows_p.at[g * L + j], sem0)
                    pltpu.async_copy(
                        q_hbm.at[i_vec[j]], rows_q.at[g * L + j], sem1)
            pltpu.make_async_copy(
                p_hbm.at[pl.ds(0, _CHUNK)], rows_p, sem0).wait()
            pltpu.make_async_copy(
                q_hbm.at[pl.ds(0, _CHUNK)], rows_q, sem1).wait()
            for g in range(_CHUNK // L):
                s = pl.ds(cbase + g * L, L)
                slot = g * L + lane
                acc = jnp.zeros((L,), jnp.float32)
                for j in range(K):
                    col = jnp.full((L,), j, jnp.int32)
                    pv = plsc.load_gather(rows_p, [slot, col])
                    qv = plsc.load_gather(rows_q, [slot, col])
                    acc = acc + pv * qv
                out_v[s] = acc
            return carry

        lax.fori_loop(0, n_chunks, chunk_body, 0)
        for cp in bias_copies:
            cp.wait()

        def finish(g, carry):
            s = pl.ds(g * L, L)
            out_v[s] = out_v[s] + (bu_v[s] + bi_v[s] + _MU)
            return carry

        lax.fori_loop(0, b_per_w // L, finish, 0)
        pltpu.sync_copy(out_v, out_hbm.at[pl.ds(base, b_per_w)])

    return sc_kernel


def kernel(user_idx, item_idx, P, Q, b_u, b_i):
    B = user_idx.shape[0]
    K = P.shape[1]
    sc_kernel = _build_sc_kernel(B, K)
    return sc_kernel(user_idx.astype(jnp.int32), item_idx.astype(jnp.int32),
                     P, Q, b_u.reshape(-1), b_i.reshape(-1))


# speedup vs baseline: 1.0156x; 1.0156x over previous
import functools

import jax
import jax.numpy as jnp
from jax import lax
from jax.experimental import pallas as pl
from jax.experimental.pallas import tpu as pltpu
from jax.experimental.pallas import tpu_sc as plsc

_MU = 3.5


@functools.lru_cache(maxsize=None)
def _build_sc_kernel(B, K):
    info = plsc.get_sparse_core_info()
    NC, NS, L = info.num_cores, info.num_subcores, info.num_lanes
    NW = NC * NS
    assert B % (8 * NW) == 0 and K % L == 0
    b_per_w = B // NW
    mesh = plsc.VectorSubcoreMesh(core_axis_name="c", subcore_axis_name="s",
                                  num_cores=NC, num_subcores=NS)

    @functools.partial(
        pl.kernel,
        out_type=jax.ShapeDtypeStruct((B,), jnp.float32),
        mesh=mesh,
        scratch_types=[
            pltpu.VMEM((b_per_w,), jnp.int32),
            pltpu.VMEM((b_per_w,), jnp.int32),
            pltpu.VMEM((b_per_w // 2, K), jnp.float32),
            pltpu.VMEM((b_per_w // 2, K), jnp.float32),
            pltpu.VMEM((b_per_w,), jnp.float32),
            pltpu.VMEM((b_per_w,), jnp.float32),
            pltpu.VMEM((b_per_w,), jnp.float32),
            pltpu.SemaphoreType.DMA,
            pltpu.SemaphoreType.DMA,
            pltpu.SemaphoreType.DMA,
            pltpu.SemaphoreType.DMA,
        ],
        compiler_params=pltpu.CompilerParams(needs_layout_passes=False),
        interpret=False,
    )
    def sc_kernel(uidx_hbm, iidx_hbm, p_hbm, q_hbm, bu_hbm, bi_hbm, out_hbm,
                  idx_u, idx_i, rows_p, rows_q, bu_v, bi_v, out_v,
                  sem0, sem1, sem2, sem3):
        wid = lax.axis_index("s") * NC + lax.axis_index("c")
        base = wid * b_per_w
        for c in range(b_per_w // 128):
            pltpu.sync_copy(uidx_hbm.at[pl.ds(base + c * 128, 128)],
                            idx_u.at[pl.ds(c * 128, 128)])
            pltpu.sync_copy(iidx_hbm.at[pl.ds(base + c * 128, 128)],
                            idx_i.at[pl.ds(c * 128, 128)])

        bias_copies = []
        for c in range(b_per_w // 128):
            sl = pl.ds(c * 128, 128)
            bias_copies.append(
                pltpu.async_copy(bu_hbm.at[idx_u.at[sl]], bu_v.at[sl], sem2))
            bias_copies.append(
                pltpu.async_copy(bi_hbm.at[idx_i.at[sl]], bi_v.at[sl], sem3))

        half = b_per_w // 2
        lane = lax.iota(jnp.int32, L)
        for h in range(2):
            hbase = h * half
            for g in range(half // L):
                u_vec = idx_u[pl.ds(hbase + g * L, L)]
                i_vec = idx_i[pl.ds(hbase + g * L, L)]
                for j in range(L):
                    pltpu.async_copy(
                        p_hbm.at[u_vec[j]], rows_p.at[g * L + j], sem0)
                    pltpu.async_copy(
                        q_hbm.at[i_vec[j]], rows_q.at[g * L + j], sem1)
            pltpu.make_async_copy(
                p_hbm.at[pl.ds(0, half)], rows_p, sem0).wait()
            pltpu.make_async_copy(
                q_hbm.at[pl.ds(0, half)], rows_q, sem1).wait()

            def group_body(g, carry):
                s = pl.ds(hbase + g * L, L)
                slot = g * L + lane
                acc = jnp.zeros((L,), jnp.float32)
                for j in range(K):
                    col = jnp.full((L,), j, jnp.int32)
                    pv = plsc.load_gather(rows_p, [slot, col])
                    qv = plsc.load_gather(rows_q, [slot, col])
                    acc = acc + pv * qv
                out_v[s] = acc
                return carry

            lax.fori_loop(0, half // L, group_body, 0)
        for cp in bias_copies:
            cp.wait()

        def finish(g, carry):
            s = pl.ds(g * L, L)
            out_v[s] = out_v[s] + (bu_v[s] + bi_v[s] + _MU)
            return carry

        lax.fori_loop(0, b_per_w // L, finish, 0)
        pltpu.sync_copy(out_v, out_hbm.at[pl.ds(base, b_per_w)])

    return sc_kernel


def kernel(user_idx, item_idx, P, Q, b_u, b_i):
    B = user_idx.shape[0]
    K = P.shape[1]
    sc_kernel = _build_sc_kernel(B, K)
    return sc_kernel(user_idx.astype(jnp.int32), item_idx.astype(jnp.int32),
                     P, Q, b_u.reshape(-1), b_i.reshape(-1))
